# Initial kernel scaffold; baseline (speedup 1.0000x reference)
import jax
import jax.numpy as jnp
from jax.experimental import pallas as pl


def kernel(x, edge_index, W0, att_src0, att_dst0, bias0, W1, att_src1, att_dst1, bias1, W2, att_src2, att_dst2, bias2):
    raise NotImplementedError("write your pallas kernel here")


# speedup vs baseline: 18.5940x; 18.5940x over previous
import functools

import jax
import jax.numpy as jnp
from jax import lax
from jax.experimental import pallas as pl
from jax.experimental.pallas import tpu as pltpu
from jax.experimental.pallas import tpu_sc as plsc

B = 16
N0 = 4096
LAYERS = 3
D = 64
DEG = 16
E = B * N0 * DEG
EPS_BN = 1e-5
NSUB = 16
K = 128


def _sc_mesh():
    return plsc.VectorSubcoreMesh(core_axis_name="c", subcore_axis_name="s",
                                  num_cores=2, num_subcores=16)


_SC_PARAMS = pltpu.CompilerParams(needs_layout_passes=False)


@functools.cache
def _dense_pre(n, bn):
    def body(h_ref, mu_ref, var_ref, w_ref, as_ref, ad_ref,
             hw_ref, a_s_ref, a_d_ref, c_ref, m_ref):
        i = pl.program_id(0)
        h = (h_ref[...] - mu_ref[...]) * lax.rsqrt(var_ref[...] + EPS_BN)
        hw = lax.dot_general(h, w_ref[...], (((1,), (1,)), ((), ())),
                             preferred_element_type=jnp.float32)
        hw_ref[...] = hw
        a_s = jnp.sum(hw * as_ref[...], axis=1)
        a_d = jnp.sum(hw * ad_ref[...], axis=1)
        a_s_ref[...] = a_s
        a_d_ref[...] = a_d
        ms = jnp.max(a_s)
        md = jnp.max(a_d)

        @pl.when(i == 0)
        def _():
            m_ref[0] = ms
            m_ref[1] = md

        @pl.when(i > 0)
        def _():
            m_ref[0] = jnp.maximum(m_ref[0], ms)
            m_ref[1] = jnp.maximum(m_ref[1], md)

        craw = m_ref[0] + m_ref[1]
        c = jnp.maximum(craw, 0.2 * craw)
        c_ref[...] = jnp.full((16,), c, jnp.float32)

    return pl.pallas_call(
        body,
        grid=(n // bn,),
        in_specs=[
            pl.BlockSpec((bn, D), lambda i: (i, 0)),
            pl.BlockSpec((1, D), lambda i: (0, 0)),
            pl.BlockSpec((1, D), lambda i: (0, 0)),
            pl.BlockSpec((D, D), lambda i: (0, 0)),
            pl.BlockSpec((1, D), lambda i: (0, 0)),
            pl.BlockSpec((1, D), lambda i: (0, 0)),
        ],
        out_specs=[
            pl.BlockSpec((bn, D), lambda i: (i, 0)),
            pl.BlockSpec((bn,), lambda i: (i,)),
            pl.BlockSpec((bn,), lambda i: (i,)),
            pl.BlockSpec((16,), lambda i: (0,)),
        ],
        out_shape=[
            jax.ShapeDtypeStruct((n, D), jnp.float32),
            jax.ShapeDtypeStruct((n,), jnp.float32),
            jax.ShapeDtypeStruct((n,), jnp.float32),
            jax.ShapeDtypeStruct((16,), jnp.float32),
        ],
        scratch_shapes=[pltpu.SMEM((2,), jnp.float32)],
    )


@functools.cache
def _dense_post(n, bn):
    np_rows = n // 2

    def body(acc_ref, den_ref, hw_ref, a_s_ref, a_d_ref, c_ref, b_ref,
             hp_ref, mu_ref, var_ref, st_ref):
        i = pl.program_id(0)
        c = jnp.max(c_ref[...])
        a = a_s_ref[...] + a_d_ref[...]
        a = jnp.maximum(a, 0.2 * a)
        exs = jnp.exp(a - c)
        num = acc_ref[...] + exs[:, None] * hw_ref[...]
        den = den_ref[...] + exs + 1e-16
        out = num / den[:, None] + b_ref[...][None, :]
        out = jnp.maximum(out, 0.0)
        hp = jnp.max(out.reshape(bn // 2, 2, D), axis=1)
        hp_ref[...] = hp
        ssum = jnp.sum(hp, axis=0, keepdims=True)
        ssq = jnp.sum(hp * hp, axis=0, keepdims=True)

        @pl.when(i == 0)
        def _():
            st_ref[0:1, :] = ssum
            st_ref[1:2, :] = ssq

        @pl.when(i > 0)
        def _():
            st_ref[0:1, :] = st_ref[0:1, :] + ssum
            st_ref[1:2, :] = st_ref[1:2, :] + ssq

        mu = st_ref[0:1, :] / np_rows
        mu_ref[...] = mu
        var_ref[...] = st_ref[1:2, :] / np_rows - mu * mu

    return pl.pallas_call(
        body,
        grid=(n // bn,),
        in_specs=[
            pl.BlockSpec((bn, D), lambda i: (i, 0)),
            pl.BlockSpec((bn,), lambda i: (i,)),
            pl.BlockSpec((bn, D), lambda i: (i, 0)),
            pl.BlockSpec((bn,), lambda i: (i,)),
            pl.BlockSpec((bn,), lambda i: (i,)),
            pl.BlockSpec((16,), lambda i: (0,)),
            pl.BlockSpec((D,), lambda i: (0,)),
        ],
        out_specs=[
            pl.BlockSpec((bn // 2, D), lambda i: (i, 0)),
            pl.BlockSpec((1, D), lambda i: (0, 0)),
            pl.BlockSpec((1, D), lambda i: (0, 0)),
        ],
        out_shape=[
            jax.ShapeDtypeStruct((np_rows, D), jnp.float32),
            jax.ShapeDtypeStruct((1, D), jnp.float32),
            jax.ShapeDtypeStruct((1, D), jnp.float32),
        ],
        scratch_shapes=[pltpu.VMEM((2, D), jnp.float32)],
    )


@functools.cache
def _bn_final(n, bn):
    def body(h_ref, mu_ref, var_ref, o_ref):
        o_ref[...] = (h_ref[...] - mu_ref[...]) * lax.rsqrt(var_ref[...] + EPS_BN)

    return pl.pallas_call(
        body,
        grid=(n // bn,),
        in_specs=[
            pl.BlockSpec((bn, D), lambda i: (i, 0)),
            pl.BlockSpec((1, D), lambda i: (0, 0)),
            pl.BlockSpec((1, D), lambda i: (0, 0)),
        ],
        out_specs=pl.BlockSpec((bn, D), lambda i: (i, 0)),
        out_shape=jax.ShapeDtypeStruct((n, D), jnp.float32),
    )


@functools.cache
def _edge_prep(mask):
    bn = 131072

    def body(s_ref, d_ref, rv_ref, dl2_ref, dlf_ref):
        s = s_ref[...]
        d = d_ref[...]
        dlf = d & mask
        rv_ref[...] = lax.shift_right_logical(s, 1)
        dl2_ref[...] = lax.shift_right_logical(dlf, 1)
        dlf_ref[...] = dlf

    return pl.pallas_call(
        body,
        grid=(E // bn,),
        in_specs=[pl.BlockSpec((bn,), lambda i: (i,)),
                  pl.BlockSpec((bn,), lambda i: (i,))],
        out_specs=[pl.BlockSpec((bn,), lambda i: (i,)),
                   pl.BlockSpec((bn,), lambda i: (i,)),
                   pl.BlockSpec((bn,), lambda i: (i,))],
        out_shape=[jax.ShapeDtypeStruct((E,), jnp.int32),
                   jax.ShapeDtypeStruct((E,), jnp.int32),
                   jax.ShapeDtypeStruct((E,), jnp.int32)],
    )


@functools.cache
def _edge_kernel(Ni, npass):
    n = B * Ni
    gpass = (B // 2) // npass
    pass_nodes = gpass * Ni
    mask = pass_nodes - 1
    prows = pass_nodes // 2
    e_per_graph = N0 * DEG
    pass_edges = gpass * e_per_graph
    ept = pass_edges // NSUB
    nchunk = ept // K
    rpt = prows // NSUB
    nz = rpt // K
    drpt = pass_nodes // NSUB
    dnz = drpt // K

    @functools.partial(
        pl.kernel, mesh=_sc_mesh(),
        out_type=[
            jax.ShapeDtypeStruct((n // 2, 2 * D), jnp.float32),
            jax.ShapeDtypeStruct((n,), jnp.float32),
        ],
        scratch_types=[
            pltpu.VMEM((K,), jnp.int32),
            pltpu.VMEM((K,), jnp.int32),
            pltpu.VMEM((K,), jnp.float32),
            pltpu.VMEM((K,), jnp.float32),
            pltpu.VMEM((K,), jnp.int32),
            pltpu.VMEM((K,), jnp.int32),
            pltpu.VMEM((K,), jnp.int32),
            pltpu.VMEM((K, 2 * D), jnp.float32),
            pltpu.VMEM((K, 2 * D), jnp.float32),
            pltpu.VMEM((K, 2 * D), jnp.float32),
            pltpu.VMEM((K,), jnp.float32),
            pltpu.VMEM((pass_nodes,), jnp.float32),
            pltpu.VMEM((pass_nodes,), jnp.float32),
            pltpu.VMEM((16,), jnp.float32),
            pltpu.VMEM_SHARED((prows, 2 * D), jnp.float32),
            pltpu.VMEM_SHARED((pass_nodes,), jnp.float32),
            pltpu.SemaphoreType.DMA,
        ],
        compiler_params=_SC_PARAMS,
        name="gat_edge_n%d" % n,
    )
    def ek(s_hbm, d_hbm, w_hbm, rv_hbm, dl2_hbm, dlf_hbm,
           as_hbm, ad_hbm, c_hbm, hw2_hbm,
           acc_hbm, den_hbm,
           sv, dv, wv, exv, rv, dl2v, dlfv, rows, scaled, zbuf, zd,
           asl, adl, cv, out_sh, den_sh, sem):
        cidx = lax.axis_index("c")
        sidx = lax.axis_index("s")
        pltpu.sync_copy(c_hbm, cv)
        zero16 = jnp.zeros((16,), jnp.float32)
        for g in range(K // 16):
            zd[pl.ds(g * 16, 16)] = zero16

        def zrow(r, _):
            for fb in range(2 * D // 16):
                zbuf[r, pl.ds(fb * 16, 16)] = zero16
            return 0

        lax.fori_loop(0, K, zrow, 0)
        rb = sidx * rpt
        drb = sidx * drpt

        def one_pass(p, _):
            gbase = cidx * (B // 2) + p * gpass
            node_base = gbase * Ni
            pltpu.sync_copy(as_hbm.at[pl.ds(node_base, pass_nodes)], asl)
            pltpu.sync_copy(ad_hbm.at[pl.ds(node_base, pass_nodes)], adl)

            def zloop(z, _):
                pltpu.sync_copy(zbuf, out_sh.at[pl.ds(rb + z * K, K)])
                return 0

            lax.fori_loop(0, nz, zloop, 0)

            def zloopd(z, _):
                pltpu.sync_copy(zd, den_sh.at[pl.ds(drb + z * K, K)])
                return 0

            lax.fori_loop(0, dnz, zloopd, 0)
            plsc.subcore_barrier()

            ebase = gbase * e_per_graph + sidx * ept
            cvec = cv[...]

            def chunk(ci, _):
                off = ebase + ci * K
                pltpu.sync_copy(s_hbm.at[pl.ds(off, K)], sv)
                pltpu.sync_copy(d_hbm.at[pl.ds(off, K)], dv)
                pltpu.sync_copy(w_hbm.at[pl.ds(off, K)], wv)
                pltpu.sync_copy(rv_hbm.at[pl.ds(off, K)], rv)
                pltpu.sync_copy(dl2_hbm.at[pl.ds(off, K)], dl2v)
                pltpu.sync_copy(dlf_hbm.at[pl.ds(off, K)], dlfv)
                for g in range(K // 16):
                    s16 = sv[pl.ds(g * 16, 16)]
                    d16 = dv[pl.ds(g * 16, 16)]
                    w16 = wv[pl.ds(g * 16, 16)]
                    sl = s16 & mask
                    dlf = d16 & mask
                    av = (plsc.load_gather(asl, [sl])
                          + plsc.load_gather(adl, [dlf]))
                    av = jnp.maximum(av, 0.2 * av)
                    exv[pl.ds(g * 16, 16)] = jnp.exp(av - cvec) * w16
                pltpu.async_copy(hw2_hbm.at[rv], rows, sem).wait()

                def grp(g, _):
                    s16 = sv[pl.ds(g * 16, 16)]
                    d16 = dv[pl.ds(g * 16, 16)]
                    e16 = exv[pl.ds(g * 16, 16)]
                    for j in range(16):
                        e = g * 16 + j
                        hoff = (s16[j] & 1) * D
                        dpar = d16[j] & 1
                        ex_e = e16[j]
                        for fb in range(D // 16):
                            v = rows[e, pl.ds(hoff + fb * 16, 16)] * ex_e
                            scaled[e, pl.ds(fb * 16, 16)] = jnp.where(
                                dpar == 0, v, zero16)
                            scaled[e, pl.ds(D + fb * 16, 16)] = jnp.where(
                                dpar == 0, zero16, v)
                    return 0

                lax.fori_loop(0, K // 16, grp, 0)
                pltpu.sync_copy(scaled, out_sh.at[dl2v], add=True)
                pltpu.sync_copy(exv, den_sh.at[dlfv], add=True)
                return 0

            lax.fori_loop(0, nchunk, chunk, 0)
            plsc.subcore_barrier()

            def wb(z, _):
                dst = pl.multiple_of(node_base // 2 + rb + z * K, 128)
                pltpu.sync_copy(out_sh.at[pl.ds(rb + z * K, K)],
                                acc_hbm.at[pl.ds(dst, K)])
                return 0

            lax.fori_loop(0, nz, wb, 0)

            def wbd(z, _):
                dst = pl.multiple_of(node_base + drb + z * K, 128)
                pltpu.sync_copy(den_sh.at[pl.ds(drb + z * K, K)],
                                den_hbm.at[pl.ds(dst, K)])
                return 0

            lax.fori_loop(0, dnz, wbd, 0)
            plsc.subcore_barrier()
            return 0

        lax.fori_loop(0, npass, one_pass, 0)

    return ek


@functools.cache
def _pool_scatter(Nc, lg):
    M = B * Nc * Nc + 1024
    dump = B * Nc * Nc
    ept = E // (2 * NSUB)
    nchunk = ept // K

    @functools.partial(
        pl.kernel, mesh=_sc_mesh(),
        out_type=[
            jax.ShapeDtypeStruct((M,), jnp.int32),
            jax.ShapeDtypeStruct((E,), jnp.int32),
            jax.ShapeDtypeStruct((E,), jnp.int32),
            jax.ShapeDtypeStruct((E,), jnp.int32),
            jax.ShapeDtypeStruct((E,), jnp.float32),
        ],
        scratch_types=[
            pltpu.VMEM((K,), jnp.int32),
            pltpu.VMEM((K,), jnp.int32),
            pltpu.VMEM((K,), jnp.float32),
            pltpu.VMEM((K,), jnp.int32),
            pltpu.VMEM((K,), jnp.int32),
            pltpu.VMEM((K,), jnp.int32),
            pltpu.VMEM((K,), jnp.int32),
            pltpu.VMEM((K,), jnp.float32),
        ],
        compiler_params=_SC_PARAMS,
        name="pool_scatter_nc%d" % Nc,
    )
    def d1(s_hbm, d_hbm, w_hbm, t_hbm, code_hbm, cs_hbm, cd_hbm, vm_hbm,
           sv, dv, wv, codev, eidv, csv, cdv, vmv):
        cidx = lax.axis_index("c")
        sidx = lax.axis_index("s")
        wid = cidx * NSUB + sidx

        def chunk(ci, _):
            off = wid * ept + ci * K
            pltpu.sync_copy(s_hbm.at[pl.ds(off, K)], sv)
            pltpu.sync_copy(d_hbm.at[pl.ds(off, K)], dv)
            pltpu.sync_copy(w_hbm.at[pl.ds(off, K)], wv)
            for g in range(K // 16):
                lanes = lax.iota(jnp.int32, 16) + g * 16
                s16 = sv[pl.ds(g * 16, 16)]
                d16 = dv[pl.ds(g * 16, 16)]
                w16 = wv[pl.ds(g * 16, 16)]
                cs = lax.shift_right_logical(s16, 1)
                cd = lax.shift_right_logical(d16, 1)
                eid = lanes + off
                valid = (w16 > 0.0) & (cs != cd)
                gb = lax.shift_right_logical(cs, lg)
                code = (lax.shift_left(gb, 2 * lg)
                        + lax.shift_left(cs & (Nc - 1), lg)
                        + (cd & (Nc - 1)))
                code = jnp.where(valid, code, dump + (eid & 1023))
                codev[pl.ds(g * 16, 16)] = code
                eidv[pl.ds(g * 16, 16)] = eid
                csv[pl.ds(g * 16, 16)] = cs
                cdv[pl.ds(g * 16, 16)] = cd
                vmv[pl.ds(g * 16, 16)] = jnp.where(valid, 1.0, 0.0).astype(
                    jnp.float32)
            pltpu.sync_copy(eidv, t_hbm.at[codev])
            pltpu.sync_copy(codev, code_hbm.at[pl.ds(off, K)])
            pltpu.sync_copy(csv, cs_hbm.at[pl.ds(off, K)])
            pltpu.sync_copy(cdv, cd_hbm.at[pl.ds(off, K)])
            pltpu.sync_copy(vmv, vm_hbm.at[pl.ds(off, K)])
            return 0

        lax.fori_loop(0, nchunk, chunk, 0)

    return d1


@functools.cache
def _pool_gather(Nc):
    M = B * Nc * Nc + 1024
    ept = E // (2 * NSUB)
    nchunk = ept // K

    @functools.partial(
        pl.kernel, mesh=_sc_mesh(),
        out_type=jax.ShapeDtypeStruct((E,), jnp.float32),
        scratch_types=[
            pltpu.VMEM((K,), jnp.int32),
            pltpu.VMEM((K,), jnp.int32),
            pltpu.VMEM((K,), jnp.float32),
            pltpu.VMEM((K,), jnp.float32),
            pltpu.SemaphoreType.DMA,
        ],
        compiler_params=_SC_PARAMS,
        name="pool_gather_nc%d" % Nc,
    )
    def d2(code_hbm, vm_hbm, t_hbm, w2_hbm, codev, tv, vmv, w2v, sem):
        cidx = lax.axis_index("c")
        sidx = lax.axis_index("s")
        wid = cidx * NSUB + sidx

        def chunk(ci, _):
            off = wid * ept + ci * K
            pltpu.sync_copy(code_hbm.at[pl.ds(off, K)], codev)
            pltpu.sync_copy(vm_hbm.at[pl.ds(off, K)], vmv)
            pltpu.async_copy(t_hbm.at[codev], tv, sem).wait()
            for g in range(K // 16):
                lanes = lax.iota(jnp.int32, 16) + g * 16
                got = tv[pl.ds(g * 16, 16)]
                vm16 = vmv[pl.ds(g * 16, 16)]
                eid = lanes + off
                keep = jnp.where(got == eid, vm16, 0.0)
                w2v[pl.ds(g * 16, 16)] = keep
            pltpu.sync_copy(w2v, w2_hbm.at[pl.ds(off, K)])
            return 0

        lax.fori_loop(0, nchunk, chunk, 0)

    return d2


def kernel(x, edge_index, W0, att_src0, att_dst0, bias0,
           W1, att_src1, att_dst1, bias1, W2, att_src2, att_dst2, bias2):
    Ws = [W0, W1, W2]
    As = [att_src0, att_src1, att_src2]
    Ad = [att_dst0, att_dst1, att_dst2]
    Bs = [bias0, bias1, bias2]

    s = edge_index[0].astype(jnp.int32)
    d = edge_index[1].astype(jnp.int32)
    w = jnp.ones((E,), jnp.float32)
    h = x
    mu = jnp.zeros((1, D), jnp.float32)
    var = jnp.full((1, D), 1.0 - EPS_BN, jnp.float32)

    Ni = N0
    npass_by_layer = [4, 2, 1]
    for i in range(LAYERS):
        n = B * Ni
        bn = min(4096, n)
        hw, a_s, a_d, cvec = _dense_pre(n, bn)(h, mu, var, Ws[i], As[i], Ad[i])
        hw2 = hw.reshape(n // 2, 2 * D)
        rv, dl2, dlf = _edge_prep(8191)(s, d)
        acc2, den = _edge_kernel(Ni, npass_by_layer[i])(
            s, d, w, rv, dl2, dlf, a_s, a_d, cvec, hw2)
        acc = acc2.reshape(n, D)
        hp, mu, var = _dense_post(n, bn)(acc, den, hw, a_s, a_d, cvec, Bs[i])
        Nc = Ni // 2
        if i < LAYERS - 1:
            lg = Nc.bit_length() - 1
            t, code, cs, cd, vm = _pool_scatter(Nc, lg)(s, d, w)
            w = _pool_gather(Nc)(code, vm, t)
            s, d = cs, cd
        h = hp
        Ni = Nc

    n_final = B * Ni
    hn = _bn_final(n_final, min(4096, n_final))(h, mu, var)
    return hn.reshape(B, Ni * D)


# speedup vs baseline: 25.0984x; 1.3498x over previous
import functools

import jax
import jax.numpy as jnp
from jax import lax
from jax.experimental import pallas as pl
from jax.experimental.pallas import tpu as pltpu
from jax.experimental.pallas import tpu_sc as plsc

B = 16
N0 = 4096
LAYERS = 3
D = 64
DEG = 16
E = B * N0 * DEG
EPS_BN = 1e-5
NSUB = 16
K = 128


def _sc_mesh():
    return plsc.VectorSubcoreMesh(core_axis_name="c", subcore_axis_name="s",
                                  num_cores=2, num_subcores=16)


_SC_PARAMS = pltpu.CompilerParams(needs_layout_passes=False)


@functools.cache
def _dense_pre(n, bn):
    def body(h_ref, mu_ref, var_ref, w_ref, as_ref, ad_ref,
             hw_ref, a_s_ref, a_d_ref, c_ref, m_ref):
        i = pl.program_id(0)
        h = (h_ref[...] - mu_ref[...]) * lax.rsqrt(var_ref[...] + EPS_BN)
        hw = lax.dot_general(h, w_ref[...], (((1,), (1,)), ((), ())),
                             preferred_element_type=jnp.float32)
        hw_ref[...] = hw
        a_s = jnp.sum(hw * as_ref[...], axis=1)
        a_d = jnp.sum(hw * ad_ref[...], axis=1)
        a_s_ref[...] = a_s
        a_d_ref[...] = a_d
        ms = jnp.max(a_s)
        md = jnp.max(a_d)

        @pl.when(i == 0)
        def _():
            m_ref[0] = ms
            m_ref[1] = md

        @pl.when(i > 0)
        def _():
            m_ref[0] = jnp.maximum(m_ref[0], ms)
            m_ref[1] = jnp.maximum(m_ref[1], md)

        craw = m_ref[0] + m_ref[1]
        c = jnp.maximum(craw, 0.2 * craw)
        c_ref[...] = jnp.full((16,), c, jnp.float32)

    return pl.pallas_call(
        body,
        grid=(n // bn,),
        in_specs=[
            pl.BlockSpec((bn, D), lambda i: (i, 0)),
            pl.BlockSpec((1, D), lambda i: (0, 0)),
            pl.BlockSpec((1, D), lambda i: (0, 0)),
            pl.BlockSpec((D, D), lambda i: (0, 0)),
            pl.BlockSpec((1, D), lambda i: (0, 0)),
            pl.BlockSpec((1, D), lambda i: (0, 0)),
        ],
        out_specs=[
            pl.BlockSpec((bn, D), lambda i: (i, 0)),
            pl.BlockSpec((bn,), lambda i: (i,)),
            pl.BlockSpec((bn,), lambda i: (i,)),
            pl.BlockSpec((16,), lambda i: (0,)),
        ],
        out_shape=[
            jax.ShapeDtypeStruct((n, D), jnp.float32),
            jax.ShapeDtypeStruct((n,), jnp.float32),
            jax.ShapeDtypeStruct((n,), jnp.float32),
            jax.ShapeDtypeStruct((16,), jnp.float32),
        ],
        scratch_shapes=[pltpu.SMEM((2,), jnp.float32)],
    )


@functools.cache
def _dense_post(n, bn):
    np_rows = n // 2

    def body(acc_ref, den_ref, hw_ref, a_s_ref, a_d_ref, c_ref, b_ref,
             hp_ref, mu_ref, var_ref, st_ref):
        i = pl.program_id(0)
        c = jnp.max(c_ref[...])
        a = a_s_ref[...] + a_d_ref[...]
        a = jnp.maximum(a, 0.2 * a)
        exs = jnp.exp(a - c)
        num = acc_ref[...] + exs[:, None] * hw_ref[...]
        den = den_ref[...] + exs + 1e-16
        out = num / den[:, None] + b_ref[...][None, :]
        out = jnp.maximum(out, 0.0)
        hp = jnp.max(out.reshape(bn // 2, 2, D), axis=1)
        hp_ref[...] = hp
        ssum = jnp.sum(hp, axis=0, keepdims=True)
        ssq = jnp.sum(hp * hp, axis=0, keepdims=True)

        @pl.when(i == 0)
        def _():
            st_ref[0:1, :] = ssum
            st_ref[1:2, :] = ssq

        @pl.when(i > 0)
        def _():
            st_ref[0:1, :] = st_ref[0:1, :] + ssum
            st_ref[1:2, :] = st_ref[1:2, :] + ssq

        mu = st_ref[0:1, :] / np_rows
        mu_ref[...] = mu
        var_ref[...] = st_ref[1:2, :] / np_rows - mu * mu

    return pl.pallas_call(
        body,
        grid=(n // bn,),
        in_specs=[
            pl.BlockSpec((bn, D), lambda i: (i, 0)),
            pl.BlockSpec((bn,), lambda i: (i,)),
            pl.BlockSpec((bn, D), lambda i: (i, 0)),
            pl.BlockSpec((bn,), lambda i: (i,)),
            pl.BlockSpec((bn,), lambda i: (i,)),
            pl.BlockSpec((16,), lambda i: (0,)),
            pl.BlockSpec((D,), lambda i: (0,)),
        ],
        out_specs=[
            pl.BlockSpec((bn // 2, D), lambda i: (i, 0)),
            pl.BlockSpec((1, D), lambda i: (0, 0)),
            pl.BlockSpec((1, D), lambda i: (0, 0)),
        ],
        out_shape=[
            jax.ShapeDtypeStruct((np_rows, D), jnp.float32),
            jax.ShapeDtypeStruct((1, D), jnp.float32),
            jax.ShapeDtypeStruct((1, D), jnp.float32),
        ],
        scratch_shapes=[pltpu.VMEM((2, D), jnp.float32)],
    )


@functools.cache
def _bn_final(n, bn):
    def body(h_ref, mu_ref, var_ref, o_ref):
        o_ref[...] = (h_ref[...] - mu_ref[...]) * lax.rsqrt(var_ref[...] + EPS_BN)

    return pl.pallas_call(
        body,
        grid=(n // bn,),
        in_specs=[
            pl.BlockSpec((bn, D), lambda i: (i, 0)),
            pl.BlockSpec((1, D), lambda i: (0, 0)),
            pl.BlockSpec((1, D), lambda i: (0, 0)),
        ],
        out_specs=pl.BlockSpec((bn, D), lambda i: (i, 0)),
        out_shape=jax.ShapeDtypeStruct((n, D), jnp.float32),
    )


@functools.cache
def _edge_prep(mask):
    bn = 131072

    def body(s_ref, d_ref, rv_ref, dl2_ref, dlf_ref):
        s = s_ref[...]
        d = d_ref[...]
        dlf = d & mask
        rv_ref[...] = lax.shift_right_logical(s, 1)
        dl2_ref[...] = lax.shift_right_logical(dlf, 1)
        dlf_ref[...] = dlf

    return pl.pallas_call(
        body,
        grid=(E // bn,),
        in_specs=[pl.BlockSpec((bn,), lambda i: (i,)),
                  pl.BlockSpec((bn,), lambda i: (i,))],
        out_specs=[pl.BlockSpec((bn,), lambda i: (i,)),
                   pl.BlockSpec((bn,), lambda i: (i,)),
                   pl.BlockSpec((bn,), lambda i: (i,))],
        out_shape=[jax.ShapeDtypeStruct((E,), jnp.int32),
                   jax.ShapeDtypeStruct((E,), jnp.int32),
                   jax.ShapeDtypeStruct((E,), jnp.int32)],
    )


@functools.cache
def _edge_kernel(Ni, npass):
    n = B * Ni
    gpass = (B // 2) // npass
    pass_nodes = gpass * Ni
    mask = pass_nodes - 1
    prows = pass_nodes // 2
    e_per_graph = N0 * DEG
    pass_edges = gpass * e_per_graph
    ept = pass_edges // NSUB
    nchunk = ept // K
    rpt = prows // NSUB
    nz = rpt // K
    drpt = pass_nodes // NSUB
    dnz = drpt // K

    @functools.partial(
        pl.kernel, mesh=_sc_mesh(),
        out_type=[
            jax.ShapeDtypeStruct((n // 2, 2 * D), jnp.float32),
            jax.ShapeDtypeStruct((n,), jnp.float32),
        ],
        scratch_types=[
            pltpu.VMEM((K,), jnp.int32),
            pltpu.VMEM((K,), jnp.int32),
            pltpu.VMEM((K,), jnp.float32),
            pltpu.VMEM((K,), jnp.float32),
            pltpu.VMEM((K,), jnp.int32),
            pltpu.VMEM((K,), jnp.int32),
            pltpu.VMEM((K,), jnp.int32),
            pltpu.VMEM((K, 2 * D), jnp.float32),
            pltpu.VMEM((K, 2 * D), jnp.float32),
            pltpu.VMEM((K, 2 * D), jnp.float32),
            pltpu.VMEM((K,), jnp.float32),
            pltpu.VMEM((pass_nodes,), jnp.float32),
            pltpu.VMEM((pass_nodes,), jnp.float32),
            pltpu.VMEM((16,), jnp.float32),
            pltpu.VMEM_SHARED((prows, 2 * D), jnp.float32),
            pltpu.VMEM_SHARED((pass_nodes,), jnp.float32),
            pltpu.SemaphoreType.DMA,
        ],
        compiler_params=_SC_PARAMS,
        name="gat_edge_n%d" % n,
    )
    def ek(s_hbm, d_hbm, w_hbm, rv_hbm, dl2_hbm, dlf_hbm,
           as_hbm, ad_hbm, c_hbm, hw2_hbm,
           acc_hbm, den_hbm,
           sv, dv, wv, exv, rv, dl2v, dlfv, rows, scaled, zbuf, zd,
           asl, adl, cv, out_sh, den_sh, sem):
        cidx = lax.axis_index("c")
        sidx = lax.axis_index("s")
        pltpu.sync_copy(c_hbm, cv)
        zero16 = jnp.zeros((16,), jnp.float32)
        for g in range(K // 16):
            zd[pl.ds(g * 16, 16)] = zero16

        def zrow(r, _):
            for fb in range(2 * D // 16):
                zbuf[r, pl.ds(fb * 16, 16)] = zero16
            return 0

        lax.fori_loop(0, K, zrow, 0)
        rb = sidx * rpt
        drb = sidx * drpt

        def one_pass(p, _):
            gbase = cidx * (B // 2) + p * gpass
            node_base = gbase * Ni
            pltpu.sync_copy(as_hbm.at[pl.ds(node_base, pass_nodes)], asl)
            pltpu.sync_copy(ad_hbm.at[pl.ds(node_base, pass_nodes)], adl)

            def zloop(z, _):
                pltpu.sync_copy(zbuf, out_sh.at[pl.ds(rb + z * K, K)])
                return 0

            lax.fori_loop(0, nz, zloop, 0)

            def zloopd(z, _):
                pltpu.sync_copy(zd, den_sh.at[pl.ds(drb + z * K, K)])
                return 0

            lax.fori_loop(0, dnz, zloopd, 0)
            plsc.subcore_barrier()

            ebase = gbase * e_per_graph + sidx * ept
            cvec = cv[...]

            def chunk(ci, _):
                off = ebase + ci * K
                cps = [
                    pltpu.async_copy(s_hbm.at[pl.ds(off, K)], sv, sem),
                    pltpu.async_copy(d_hbm.at[pl.ds(off, K)], dv, sem),
                    pltpu.async_copy(w_hbm.at[pl.ds(off, K)], wv, sem),
                    pltpu.async_copy(rv_hbm.at[pl.ds(off, K)], rv, sem),
                    pltpu.async_copy(dl2_hbm.at[pl.ds(off, K)], dl2v, sem),
                    pltpu.async_copy(dlf_hbm.at[pl.ds(off, K)], dlfv, sem),
                ]
                for cp in cps:
                    cp.wait()
                for g in range(K // 16):
                    s16 = sv[pl.ds(g * 16, 16)]
                    d16 = dv[pl.ds(g * 16, 16)]
                    w16 = wv[pl.ds(g * 16, 16)]
                    sl = s16 & mask
                    dlf = d16 & mask
                    av = (plsc.load_gather(asl, [sl])
                          + plsc.load_gather(adl, [dlf]))
                    av = jnp.maximum(av, 0.2 * av)
                    exv[pl.ds(g * 16, 16)] = jnp.exp(av - cvec) * w16
                pltpu.async_copy(hw2_hbm.at[rv], rows, sem).wait()

                def grp(g, _):
                    s16 = sv[pl.ds(g * 16, 16)]
                    d16 = dv[pl.ds(g * 16, 16)]
                    e16 = exv[pl.ds(g * 16, 16)]
                    for j in range(16):
                        e = g * 16 + j
                        hoff = (s16[j] & 1) * D
                        dpar = d16[j] & 1
                        ex_e = e16[j]
                        for fb in range(D // 16):
                            v = rows[e, pl.ds(hoff + fb * 16, 16)] * ex_e
                            scaled[e, pl.ds(fb * 16, 16)] = jnp.where(
                                dpar == 0, v, zero16)
                            scaled[e, pl.ds(D + fb * 16, 16)] = jnp.where(
                                dpar == 0, zero16, v)
                    return 0

                lax.fori_loop(0, K // 16, grp, 0)
                pltpu.sync_copy(scaled, out_sh.at[dl2v], add=True)
                pltpu.sync_copy(exv, den_sh.at[dlfv], add=True)
                return 0

            lax.fori_loop(0, nchunk, chunk, 0)
            plsc.subcore_barrier()

            def wb(z, _):
                dst = pl.multiple_of(node_base // 2 + rb + z * K, 128)
                pltpu.sync_copy(out_sh.at[pl.ds(rb + z * K, K)],
                                acc_hbm.at[pl.ds(dst, K)])
                return 0

            lax.fori_loop(0, nz, wb, 0)

            def wbd(z, _):
                dst = pl.multiple_of(node_base + drb + z * K, 128)
                pltpu.sync_copy(den_sh.at[pl.ds(drb + z * K, K)],
                                den_hbm.at[pl.ds(dst, K)])
                return 0

            lax.fori_loop(0, dnz, wbd, 0)
            plsc.subcore_barrier()
            return 0

        lax.fori_loop(0, npass, one_pass, 0)

    return ek


KS = 2048


@functools.cache
def _pool_scatter(Nc, lg):
    M = B * Nc * Nc + 1024
    dump = B * Nc * Nc
    ept = E // (2 * NSUB)
    nchunk = ept // KS
    nrow = KS // K

    @functools.partial(
        pl.kernel, mesh=_sc_mesh(),
        out_type=[
            jax.ShapeDtypeStruct((M,), jnp.int32),
            jax.ShapeDtypeStruct((E // K, K), jnp.int32),
            jax.ShapeDtypeStruct((E,), jnp.int32),
            jax.ShapeDtypeStruct((E,), jnp.int32),
            jax.ShapeDtypeStruct((E,), jnp.float32),
        ],
        scratch_types=[
            pltpu.VMEM((KS,), jnp.int32),
            pltpu.VMEM((KS,), jnp.int32),
            pltpu.VMEM((KS,), jnp.float32),
            pltpu.VMEM((nrow, K), jnp.int32),
            pltpu.VMEM((nrow, K), jnp.int32),
            pltpu.VMEM((KS,), jnp.int32),
            pltpu.VMEM((KS,), jnp.int32),
            pltpu.VMEM((KS,), jnp.float32),
            pltpu.SemaphoreType.DMA,
        ],
        compiler_params=_SC_PARAMS,
        name="pool_scatter_nc%d" % Nc,
    )
    def d1(s_hbm, d_hbm, w_hbm, t_hbm, code_hbm, cs_hbm, cd_hbm, vm_hbm,
           sv, dv, wv, codev, eidv, csv, cdv, vmv, sem):
        cidx = lax.axis_index("c")
        sidx = lax.axis_index("s")
        wid = cidx * NSUB + sidx

        def chunk(ci, _):
            off = wid * ept + ci * KS
            cps = [
                pltpu.async_copy(s_hbm.at[pl.ds(off, KS)], sv, sem),
                pltpu.async_copy(d_hbm.at[pl.ds(off, KS)], dv, sem),
                pltpu.async_copy(w_hbm.at[pl.ds(off, KS)], wv, sem),
            ]
            for cp in cps:
                cp.wait()

            def vrow(r, _):
                for cgrp in range(K // 16):
                    base = r * K + cgrp * 16
                    s16 = sv[pl.ds(base, 16)]
                    d16 = dv[pl.ds(base, 16)]
                    w16 = wv[pl.ds(base, 16)]
                    cs = lax.shift_right_logical(s16, 1)
                    cd = lax.shift_right_logical(d16, 1)
                    eid = lax.iota(jnp.int32, 16) + (off + base)
                    valid = (w16 > 0.0) & (cs != cd)
                    gb = lax.shift_right_logical(cs, lg)
                    code = (lax.shift_left(gb, 2 * lg)
                            + lax.shift_left(cs & (Nc - 1), lg)
                            + (cd & (Nc - 1)))
                    code = jnp.where(valid, code, dump + (eid & 1023))
                    codev[r, pl.ds(cgrp * 16, 16)] = code
                    eidv[r, pl.ds(cgrp * 16, 16)] = eid
                    csv[pl.ds(base, 16)] = cs
                    cdv[pl.ds(base, 16)] = cd
                    vmv[pl.ds(base, 16)] = jnp.where(valid, 1.0, 0.0).astype(
                        jnp.float32)
                return 0

            lax.fori_loop(0, nrow, vrow, 0)
            scps = [pltpu.async_copy(eidv.at[j], t_hbm.at[codev.at[j]], sem)
                    for j in range(nrow)]
            scps.append(pltpu.async_copy(
                codev, code_hbm.at[pl.ds(pl.multiple_of(off // K, 16), nrow)],
                sem))
            scps.append(pltpu.async_copy(csv, cs_hbm.at[pl.ds(off, KS)], sem))
            scps.append(pltpu.async_copy(cdv, cd_hbm.at[pl.ds(off, KS)], sem))
            scps.append(pltpu.async_copy(vmv, vm_hbm.at[pl.ds(off, KS)], sem))
            for cp in scps:
                cp.wait()
            return 0

        lax.fori_loop(0, nchunk, chunk, 0)

    return d1


@functools.cache
def _pool_gather(Nc):
    M = B * Nc * Nc + 1024
    ept = E // (2 * NSUB)
    nchunk = ept // KS
    nrow = KS // K

    @functools.partial(
        pl.kernel, mesh=_sc_mesh(),
        out_type=jax.ShapeDtypeStruct((E,), jnp.float32),
        scratch_types=[
            pltpu.VMEM((nrow, K), jnp.int32),
            pltpu.VMEM((nrow, K), jnp.int32),
            pltpu.VMEM((KS,), jnp.float32),
            pltpu.VMEM((KS,), jnp.float32),
            pltpu.SemaphoreType.DMA,
        ],
        compiler_params=_SC_PARAMS,
        name="pool_gather_nc%d" % Nc,
    )
    def d2(code_hbm, vm_hbm, t_hbm, w2_hbm, codev, tv, vmv, w2v, sem):
        cidx = lax.axis_index("c")
        sidx = lax.axis_index("s")
        wid = cidx * NSUB + sidx

        def chunk(ci, _):
            off = wid * ept + ci * KS
            cps = [
                pltpu.async_copy(
                    code_hbm.at[pl.ds(pl.multiple_of(off // K, 16), nrow)],
                    codev, sem),
                pltpu.async_copy(vm_hbm.at[pl.ds(off, KS)], vmv, sem),
            ]
            for cp in cps:
                cp.wait()
            gcps = [pltpu.async_copy(t_hbm.at[codev.at[j]], tv.at[j], sem)
                    for j in range(nrow)]
            for cp in gcps:
                cp.wait()

            def vrow(r, _):
                for cgrp in range(K // 16):
                    base = r * K + cgrp * 16
                    got = tv[r, pl.ds(cgrp * 16, 16)]
                    vm16 = vmv[pl.ds(base, 16)]
                    eid = lax.iota(jnp.int32, 16) + (off + base)
                    w2v[pl.ds(base, 16)] = jnp.where(got == eid, vm16, 0.0)
                return 0

            lax.fori_loop(0, nrow, vrow, 0)
            pltpu.sync_copy(w2v, w2_hbm.at[pl.ds(off, KS)])
            return 0

        lax.fori_loop(0, nchunk, chunk, 0)

    return d2


def kernel(x, edge_index, W0, att_src0, att_dst0, bias0,
           W1, att_src1, att_dst1, bias1, W2, att_src2, att_dst2, bias2):
    Ws = [W0, W1, W2]
    As = [att_src0, att_src1, att_src2]
    Ad = [att_dst0, att_dst1, att_dst2]
    Bs = [bias0, bias1, bias2]

    s = edge_index[0].astype(jnp.int32)
    d = edge_index[1].astype(jnp.int32)
    w = jnp.ones((E,), jnp.float32)
    h = x
    mu = jnp.zeros((1, D), jnp.float32)
    var = jnp.full((1, D), 1.0 - EPS_BN, jnp.float32)

    Ni = N0
    npass_by_layer = [4, 2, 1]
    for i in range(LAYERS):
        n = B * Ni
        bn = min(4096, n)
        hw, a_s, a_d, cvec = _dense_pre(n, bn)(h, mu, var, Ws[i], As[i], Ad[i])
        hw2 = hw.reshape(n // 2, 2 * D)
        rv, dl2, dlf = _edge_prep(8191)(s, d)
        acc2, den = _edge_kernel(Ni, npass_by_layer[i])(
            s, d, w, rv, dl2, dlf, a_s, a_d, cvec, hw2)
        acc = acc2.reshape(n, D)
        hp, mu, var = _dense_post(n, bn)(acc, den, hw, a_s, a_d, cvec, Bs[i])
        Nc = Ni // 2
        if i < LAYERS - 1:
            lg = Nc.bit_length() - 1
            t, code, cs, cd, vm = _pool_scatter(Nc, lg)(s, d, w)
            w = _pool_gather(Nc)(code, vm, t)
            s, d = cs, cd
        h = hp
        Ni = Nc

    n_final = B * Ni
    hn = _bn_final(n_final, min(4096, n_final))(h, mu, var)
    return hn.reshape(B, Ni * D)


# speedup vs baseline: 25.6503x; 1.0220x over previous
import functools

import jax
import jax.numpy as jnp
from jax import lax
from jax.experimental import pallas as pl
from jax.experimental.pallas import tpu as pltpu
from jax.experimental.pallas import tpu_sc as plsc

B = 16
N0 = 4096
LAYERS = 3
D = 64
DEG = 16
E = B * N0 * DEG
EPS_BN = 1e-5
NSUB = 16
K = 128


def _sc_mesh():
    return plsc.VectorSubcoreMesh(core_axis_name="c", subcore_axis_name="s",
                                  num_cores=2, num_subcores=16)


_SC_PARAMS = pltpu.CompilerParams(needs_layout_passes=False)


@functools.cache
def _dense_pre(n, bn):
    def body(h_ref, mu_ref, var_ref, w_ref, as_ref, ad_ref,
             hw_ref, a_s_ref, a_d_ref, c_ref, m_ref):
        i = pl.program_id(0)
        h = (h_ref[...] - mu_ref[...]) * lax.rsqrt(var_ref[...] + EPS_BN)
        hw = lax.dot_general(h, w_ref[...], (((1,), (1,)), ((), ())),
                             preferred_element_type=jnp.float32)
        hw_ref[...] = hw
        a_s = jnp.sum(hw * as_ref[...], axis=1)
        a_d = jnp.sum(hw * ad_ref[...], axis=1)
        a_s_ref[...] = a_s
        a_d_ref[...] = a_d
        ms = jnp.max(a_s)
        md = jnp.max(a_d)

        @pl.when(i == 0)
        def _():
            m_ref[0] = ms
            m_ref[1] = md

        @pl.when(i > 0)
        def _():
            m_ref[0] = jnp.maximum(m_ref[0], ms)
            m_ref[1] = jnp.maximum(m_ref[1], md)

        craw = m_ref[0] + m_ref[1]
        c = jnp.maximum(craw, 0.2 * craw)
        c_ref[...] = jnp.full((16,), c, jnp.float32)

    return pl.pallas_call(
        body,
        grid=(n // bn,),
        in_specs=[
            pl.BlockSpec((bn, D), lambda i: (i, 0)),
            pl.BlockSpec((1, D), lambda i: (0, 0)),
            pl.BlockSpec((1, D), lambda i: (0, 0)),
            pl.BlockSpec((D, D), lambda i: (0, 0)),
            pl.BlockSpec((1, D), lambda i: (0, 0)),
            pl.BlockSpec((1, D), lambda i: (0, 0)),
        ],
        out_specs=[
            pl.BlockSpec((bn, D), lambda i: (i, 0)),
            pl.BlockSpec((bn,), lambda i: (i,)),
            pl.BlockSpec((bn,), lambda i: (i,)),
            pl.BlockSpec((16,), lambda i: (0,)),
        ],
        out_shape=[
            jax.ShapeDtypeStruct((n, D), jnp.float32),
            jax.ShapeDtypeStruct((n,), jnp.float32),
            jax.ShapeDtypeStruct((n,), jnp.float32),
            jax.ShapeDtypeStruct((16,), jnp.float32),
        ],
        scratch_shapes=[pltpu.SMEM((2,), jnp.float32)],
    )


@functools.cache
def _dense_post(n, bn):
    np_rows = n // 2

    def body(acc_ref, den_ref, hw_ref, a_s_ref, a_d_ref, c_ref, b_ref,
             hp_ref, mu_ref, var_ref, st_ref):
        i = pl.program_id(0)
        c = jnp.max(c_ref[...])
        a = a_s_ref[...] + a_d_ref[...]
        a = jnp.maximum(a, 0.2 * a)
        exs = jnp.exp(a - c)
        num = acc_ref[...] + exs[:, None] * hw_ref[...]
        den = den_ref[...] + exs + 1e-16
        out = num / den[:, None] + b_ref[...][None, :]
        out = jnp.maximum(out, 0.0)
        hp = jnp.max(out.reshape(bn // 2, 2, D), axis=1)
        hp_ref[...] = hp
        ssum = jnp.sum(hp, axis=0, keepdims=True)
        ssq = jnp.sum(hp * hp, axis=0, keepdims=True)

        @pl.when(i == 0)
        def _():
            st_ref[0:1, :] = ssum
            st_ref[1:2, :] = ssq

        @pl.when(i > 0)
        def _():
            st_ref[0:1, :] = st_ref[0:1, :] + ssum
            st_ref[1:2, :] = st_ref[1:2, :] + ssq

        mu = st_ref[0:1, :] / np_rows
        mu_ref[...] = mu
        var_ref[...] = st_ref[1:2, :] / np_rows - mu * mu

    return pl.pallas_call(
        body,
        grid=(n // bn,),
        in_specs=[
            pl.BlockSpec((bn, D), lambda i: (i, 0)),
            pl.BlockSpec((bn,), lambda i: (i,)),
            pl.BlockSpec((bn, D), lambda i: (i, 0)),
            pl.BlockSpec((bn,), lambda i: (i,)),
            pl.BlockSpec((bn,), lambda i: (i,)),
            pl.BlockSpec((16,), lambda i: (0,)),
            pl.BlockSpec((D,), lambda i: (0,)),
        ],
        out_specs=[
            pl.BlockSpec((bn // 2, D), lambda i: (i, 0)),
            pl.BlockSpec((1, D), lambda i: (0, 0)),
            pl.BlockSpec((1, D), lambda i: (0, 0)),
        ],
        out_shape=[
            jax.ShapeDtypeStruct((np_rows, D), jnp.float32),
            jax.ShapeDtypeStruct((1, D), jnp.float32),
            jax.ShapeDtypeStruct((1, D), jnp.float32),
        ],
        scratch_shapes=[pltpu.VMEM((2, D), jnp.float32)],
    )


@functools.cache
def _bn_final(n, bn):
    def body(h_ref, mu_ref, var_ref, o_ref):
        o_ref[...] = (h_ref[...] - mu_ref[...]) * lax.rsqrt(var_ref[...] + EPS_BN)

    return pl.pallas_call(
        body,
        grid=(n // bn,),
        in_specs=[
            pl.BlockSpec((bn, D), lambda i: (i, 0)),
            pl.BlockSpec((1, D), lambda i: (0, 0)),
            pl.BlockSpec((1, D), lambda i: (0, 0)),
        ],
        out_specs=pl.BlockSpec((bn, D), lambda i: (i, 0)),
        out_shape=jax.ShapeDtypeStruct((n, D), jnp.float32),
    )


@functools.cache
def _edge_prep(mask):
    bn = 131072

    def body(s_ref, d_ref, rv_ref, dl2_ref, dlf_ref):
        s = s_ref[...]
        d = d_ref[...]
        dlf = d & mask
        rv_ref[...] = lax.shift_right_logical(s, 1)
        dl2_ref[...] = lax.shift_right_logical(dlf, 1)
        dlf_ref[...] = dlf

    return pl.pallas_call(
        body,
        grid=(E // bn,),
        in_specs=[pl.BlockSpec((bn,), lambda i: (i,)),
                  pl.BlockSpec((bn,), lambda i: (i,))],
        out_specs=[pl.BlockSpec((bn,), lambda i: (i,)),
                   pl.BlockSpec((bn,), lambda i: (i,)),
                   pl.BlockSpec((bn,), lambda i: (i,))],
        out_shape=[jax.ShapeDtypeStruct((E,), jnp.int32),
                   jax.ShapeDtypeStruct((E,), jnp.int32),
                   jax.ShapeDtypeStruct((E,), jnp.int32)],
    )


@functools.cache
def _edge_kernel(Ni, npass):
    n = B * Ni
    gpass = (B // 2) // npass
    pass_nodes = gpass * Ni
    mask = pass_nodes - 1
    prows = pass_nodes // 2
    e_per_graph = N0 * DEG
    pass_edges = gpass * e_per_graph
    ept = pass_edges // NSUB
    nchunk = ept // K
    rpt = prows // NSUB
    nz = rpt // K
    drpt = pass_nodes // NSUB
    dnz = drpt // K

    @functools.partial(
        pl.kernel, mesh=_sc_mesh(),
        out_type=[
            jax.ShapeDtypeStruct((n // 2, 2 * D), jnp.float32),
            jax.ShapeDtypeStruct((n,), jnp.float32),
        ],
        scratch_types=[
            pltpu.VMEM((K,), jnp.int32),
            pltpu.VMEM((K,), jnp.int32),
            pltpu.VMEM((K,), jnp.float32),
            pltpu.VMEM((K,), jnp.float32),
            pltpu.VMEM((K,), jnp.int32),
            pltpu.VMEM((K,), jnp.int32),
            pltpu.VMEM((K,), jnp.int32),
            pltpu.VMEM((K, 2 * D), jnp.float32),
            pltpu.VMEM((K, 2 * D), jnp.float32),
            pltpu.VMEM((K, 2 * D), jnp.float32),
            pltpu.VMEM((K,), jnp.float32),
            pltpu.VMEM((pass_nodes,), jnp.float32),
            pltpu.VMEM((pass_nodes,), jnp.float32),
            pltpu.VMEM((16,), jnp.float32),
            pltpu.VMEM_SHARED((prows, 2 * D), jnp.float32),
            pltpu.VMEM_SHARED((pass_nodes,), jnp.float32),
            pltpu.SemaphoreType.DMA,
        ],
        compiler_params=_SC_PARAMS,
        name="gat_edge_n%d" % n,
    )
    def ek(s_hbm, d_hbm, w_hbm, rv_hbm, dl2_hbm, dlf_hbm,
           as_hbm, ad_hbm, c_hbm, hw2_hbm,
           acc_hbm, den_hbm,
           sv, dv, wv, exv, rv, dl2v, dlfv, rows, scaled, zbuf, zd,
           asl, adl, cv, out_sh, den_sh, sem):
        cidx = lax.axis_index("c")
        sidx = lax.axis_index("s")
        pltpu.sync_copy(c_hbm, cv)
        zero16 = jnp.zeros((16,), jnp.float32)
        for g in range(K // 16):
            zd[pl.ds(g * 16, 16)] = zero16

        def zrow(r, _):
            for fb in range(2 * D // 16):
                zbuf[r, pl.ds(fb * 16, 16)] = zero16
            return 0

        lax.fori_loop(0, K, zrow, 0)
        rb = sidx * rpt
        drb = sidx * drpt

        def one_pass(p, _):
            gbase = cidx * (B // 2) + p * gpass
            node_base = gbase * Ni
            pltpu.sync_copy(as_hbm.at[pl.ds(node_base, pass_nodes)], asl)
            pltpu.sync_copy(ad_hbm.at[pl.ds(node_base, pass_nodes)], adl)

            def zloop(z, _):
                pltpu.sync_copy(zbuf, out_sh.at[pl.ds(rb + z * K, K)])
                return 0

            lax.fori_loop(0, nz, zloop, 0)

            def zloopd(z, _):
                pltpu.sync_copy(zd, den_sh.at[pl.ds(drb + z * K, K)])
                return 0

            lax.fori_loop(0, dnz, zloopd, 0)
            plsc.subcore_barrier()

            ebase = gbase * e_per_graph + sidx * ept
            cvec = cv[...]

            def chunk(ci, _):
                off = ebase + ci * K
                cps = [
                    pltpu.async_copy(s_hbm.at[pl.ds(off, K)], sv, sem),
                    pltpu.async_copy(d_hbm.at[pl.ds(off, K)], dv, sem),
                    pltpu.async_copy(w_hbm.at[pl.ds(off, K)], wv, sem),
                    pltpu.async_copy(rv_hbm.at[pl.ds(off, K)], rv, sem),
                    pltpu.async_copy(dl2_hbm.at[pl.ds(off, K)], dl2v, sem),
                    pltpu.async_copy(dlf_hbm.at[pl.ds(off, K)], dlfv, sem),
                ]
                for cp in cps:
                    cp.wait()
                gcp = pltpu.async_copy(hw2_hbm.at[rv], rows, sem)
                for g in range(K // 16):
                    s16 = sv[pl.ds(g * 16, 16)]
                    d16 = dv[pl.ds(g * 16, 16)]
                    w16 = wv[pl.ds(g * 16, 16)]
                    sl = s16 & mask
                    dlf = d16 & mask
                    av = (plsc.load_gather(asl, [sl])
                          + plsc.load_gather(adl, [dlf]))
                    av = jnp.maximum(av, 0.2 * av)
                    exv[pl.ds(g * 16, 16)] = jnp.exp(av - cvec) * w16
                gcp.wait()

                def grp(g, _):
                    s16 = sv[pl.ds(g * 16, 16)]
                    d16 = dv[pl.ds(g * 16, 16)]
                    e16 = exv[pl.ds(g * 16, 16)]
                    for j in range(16):
                        e = g * 16 + j
                        hoff = (s16[j] & 1) * D
                        dpar = d16[j] & 1
                        ex_e = e16[j]
                        for fb in range(D // 16):
                            v = rows[e, pl.ds(hoff + fb * 16, 16)] * ex_e
                            scaled[e, pl.ds(fb * 16, 16)] = jnp.where(
                                dpar == 0, v, zero16)
                            scaled[e, pl.ds(D + fb * 16, 16)] = jnp.where(
                                dpar == 0, zero16, v)
                    return 0

                lax.fori_loop(0, K // 16, grp, 0)
                sc1 = pltpu.async_copy(scaled, out_sh.at[dl2v], sem, add=True)
                sc2 = pltpu.async_copy(exv, den_sh.at[dlfv], sem, add=True)
                sc1.wait()
                sc2.wait()
                return 0

            lax.fori_loop(0, nchunk, chunk, 0)
            plsc.subcore_barrier()

            def wb(z, _):
                dst = pl.multiple_of(node_base // 2 + rb + z * K, 128)
                pltpu.sync_copy(out_sh.at[pl.ds(rb + z * K, K)],
                                acc_hbm.at[pl.ds(dst, K)])
                return 0

            lax.fori_loop(0, nz, wb, 0)

            def wbd(z, _):
                dst = pl.multiple_of(node_base + drb + z * K, 128)
                pltpu.sync_copy(den_sh.at[pl.ds(drb + z * K, K)],
                                den_hbm.at[pl.ds(dst, K)])
                return 0

            lax.fori_loop(0, dnz, wbd, 0)
            plsc.subcore_barrier()
            return 0

        lax.fori_loop(0, npass, one_pass, 0)

    return ek


KS = 2048


@functools.cache
def _pool_scatter(Nc, lg):
    M = B * Nc * Nc + 1024
    dump = B * Nc * Nc
    ept = E // (2 * NSUB)
    nchunk = ept // KS
    nrow = KS // K

    @functools.partial(
        pl.kernel, mesh=_sc_mesh(),
        out_type=[
            jax.ShapeDtypeStruct((M,), jnp.int32),
            jax.ShapeDtypeStruct((E // K, K), jnp.int32),
            jax.ShapeDtypeStruct((E,), jnp.int32),
            jax.ShapeDtypeStruct((E,), jnp.int32),
            jax.ShapeDtypeStruct((E,), jnp.float32),
        ],
        scratch_types=[
            pltpu.VMEM((KS,), jnp.int32),
            pltpu.VMEM((KS,), jnp.int32),
            pltpu.VMEM((KS,), jnp.float32),
            pltpu.VMEM((nrow, K), jnp.int32),
            pltpu.VMEM((nrow, K), jnp.int32),
            pltpu.VMEM((KS,), jnp.int32),
            pltpu.VMEM((KS,), jnp.int32),
            pltpu.VMEM((KS,), jnp.float32),
            pltpu.SemaphoreType.DMA,
        ],
        compiler_params=_SC_PARAMS,
        name="pool_scatter_nc%d" % Nc,
    )
    def d1(s_hbm, d_hbm, w_hbm, t_hbm, code_hbm, cs_hbm, cd_hbm, vm_hbm,
           sv, dv, wv, codev, eidv, csv, cdv, vmv, sem):
        cidx = lax.axis_index("c")
        sidx = lax.axis_index("s")
        wid = cidx * NSUB + sidx

        def chunk(ci, _):
            off = wid * ept + ci * KS
            cps = [
                pltpu.async_copy(s_hbm.at[pl.ds(off, KS)], sv, sem),
                pltpu.async_copy(d_hbm.at[pl.ds(off, KS)], dv, sem),
                pltpu.async_copy(w_hbm.at[pl.ds(off, KS)], wv, sem),
            ]
            for cp in cps:
                cp.wait()

            def vrow(r, _):
                for cgrp in range(K // 16):
                    base = r * K + cgrp * 16
                    s16 = sv[pl.ds(base, 16)]
                    d16 = dv[pl.ds(base, 16)]
                    w16 = wv[pl.ds(base, 16)]
                    cs = lax.shift_right_logical(s16, 1)
                    cd = lax.shift_right_logical(d16, 1)
                    eid = lax.iota(jnp.int32, 16) + (off + base)
                    valid = (w16 > 0.0) & (cs != cd)
                    gb = lax.shift_right_logical(cs, lg)
                    code = (lax.shift_left(gb, 2 * lg)
                            + lax.shift_left(cs & (Nc - 1), lg)
                            + (cd & (Nc - 1)))
                    code = jnp.where(valid, code, dump + (eid & 1023))
                    codev[r, pl.ds(cgrp * 16, 16)] = code
                    eidv[r, pl.ds(cgrp * 16, 16)] = eid
                    csv[pl.ds(base, 16)] = cs
                    cdv[pl.ds(base, 16)] = cd
                    vmv[pl.ds(base, 16)] = jnp.where(valid, 1.0, 0.0).astype(
                        jnp.float32)
                return 0

            lax.fori_loop(0, nrow, vrow, 0)
            scps = [pltpu.async_copy(eidv.at[j], t_hbm.at[codev.at[j]], sem)
                    for j in range(nrow)]
            scps.append(pltpu.async_copy(
                codev, code_hbm.at[pl.ds(pl.multiple_of(off // K, 16), nrow)],
                sem))
            scps.append(pltpu.async_copy(csv, cs_hbm.at[pl.ds(off, KS)], sem))
            scps.append(pltpu.async_copy(cdv, cd_hbm.at[pl.ds(off, KS)], sem))
            scps.append(pltpu.async_copy(vmv, vm_hbm.at[pl.ds(off, KS)], sem))
            for cp in scps:
                cp.wait()
            return 0

        lax.fori_loop(0, nchunk, chunk, 0)

    return d1


@functools.cache
def _pool_gather(Nc):
    M = B * Nc * Nc + 1024
    ept = E // (2 * NSUB)
    nchunk = ept // KS
    nrow = KS // K

    @functools.partial(
        pl.kernel, mesh=_sc_mesh(),
        out_type=jax.ShapeDtypeStruct((E,), jnp.float32),
        scratch_types=[
            pltpu.VMEM((nrow, K), jnp.int32),
            pltpu.VMEM((nrow, K), jnp.int32),
            pltpu.VMEM((KS,), jnp.float32),
            pltpu.VMEM((KS,), jnp.float32),
            pltpu.SemaphoreType.DMA,
        ],
        compiler_params=_SC_PARAMS,
        name="pool_gather_nc%d" % Nc,
    )
    def d2(code_hbm, vm_hbm, t_hbm, w2_hbm, codev, tv, vmv, w2v, sem):
        cidx = lax.axis_index("c")
        sidx = lax.axis_index("s")
        wid = cidx * NSUB + sidx

        def chunk(ci, _):
            off = wid * ept + ci * KS
            cps = [
                pltpu.async_copy(
                    code_hbm.at[pl.ds(pl.multiple_of(off // K, 16), nrow)],
                    codev, sem),
                pltpu.async_copy(vm_hbm.at[pl.ds(off, KS)], vmv, sem),
            ]
            for cp in cps:
                cp.wait()
            gcps = [pltpu.async_copy(t_hbm.at[codev.at[j]], tv.at[j], sem)
                    for j in range(nrow)]
            for cp in gcps:
                cp.wait()

            def vrow(r, _):
                for cgrp in range(K // 16):
                    base = r * K + cgrp * 16
                    got = tv[r, pl.ds(cgrp * 16, 16)]
                    vm16 = vmv[pl.ds(base, 16)]
                    eid = lax.iota(jnp.int32, 16) + (off + base)
                    w2v[pl.ds(base, 16)] = jnp.where(got == eid, vm16, 0.0)
                return 0

            lax.fori_loop(0, nrow, vrow, 0)
            pltpu.sync_copy(w2v, w2_hbm.at[pl.ds(off, KS)])
            return 0

        lax.fori_loop(0, nchunk, chunk, 0)

    return d2


def kernel(x, edge_index, W0, att_src0, att_dst0, bias0,
           W1, att_src1, att_dst1, bias1, W2, att_src2, att_dst2, bias2):
    Ws = [W0, W1, W2]
    As = [att_src0, att_src1, att_src2]
    Ad = [att_dst0, att_dst1, att_dst2]
    Bs = [bias0, bias1, bias2]

    s = edge_index[0].astype(jnp.int32)
    d = edge_index[1].astype(jnp.int32)
    w = jnp.ones((E,), jnp.float32)
    h = x
    mu = jnp.zeros((1, D), jnp.float32)
    var = jnp.full((1, D), 1.0 - EPS_BN, jnp.float32)

    Ni = N0
    npass_by_layer = [4, 2, 1]
    for i in range(LAYERS):
        n = B * Ni
        bn = min(4096, n)
        hw, a_s, a_d, cvec = _dense_pre(n, bn)(h, mu, var, Ws[i], As[i], Ad[i])
        hw2 = hw.reshape(n // 2, 2 * D)
        rv, dl2, dlf = _edge_prep(8191)(s, d)
        acc2, den = _edge_kernel(Ni, npass_by_layer[i])(
            s, d, w, rv, dl2, dlf, a_s, a_d, cvec, hw2)
        acc = acc2.reshape(n, D)
        hp, mu, var = _dense_post(n, bn)(acc, den, hw, a_s, a_d, cvec, Bs[i])
        Nc = Ni // 2
        if i < LAYERS - 1:
            lg = Nc.bit_length() - 1
            t, code, cs, cd, vm = _pool_scatter(Nc, lg)(s, d, w)
            w = _pool_gather(Nc)(code, vm, t)
            s, d = cs, cd
        h = hp
        Ni = Nc

    n_final = B * Ni
    hn = _bn_final(n_final, min(4096, n_final))(h, mu, var)
    return hn.reshape(B, Ni * D)
